# baseline (speedup 1.0000x reference)
import functools

import jax
import jax.numpy as jnp
import numpy as np
from jax import lax
from jax.experimental import pallas as pl
from jax.experimental.pallas import tpu as pltpu
from jax.experimental.pallas import tpu_sc as plsc

_B, _N, _CIN = 8, 4096, 16
_M = 1024
_R0, _R1 = 0.2, 0.4
_NS0, _NS1 = 16, 32
_EPS = 1e-5
_MBA = 64
_MB3 = 128
_MB2 = 128

_NC, _NSC = 2, 16
_NW = _NC * _NSC
_RPW = (_B * _M) // _NW
_RB = 16
_NBATCH = _RPW // _RB
_WPR = _N // 4 // 16

_JJ = np.arange(_N)
_PERM = (_JJ // 64) * 64 + 16 * (_JJ % 4) + (_JJ % 64) // 4


def _cumsum_lanes(x):
    lane = jax.lax.broadcasted_iota(jnp.int32, x.shape, 1)
    s = 1
    while s < x.shape[1]:
        shifted = pltpu.roll(x, s, 1)
        x = x + jnp.where(lane >= s, shifted, 0)
        s *= 2
    return x


def _k1_body(idx_ref, xyz_ref, feat_ref, xtp_ref, perm_ref, bd_ref,
             w0t_ref, w1t_ref,
             nx_ref, code_ref, stop_ref, tw0_ref, tw1_ref):
    j = pl.program_id(1)
    xyz = xyz_ref[...]

    @pl.when(j == 0)
    def _():
        dn = (((0,), (0,)), ((), ()))
        tw0_ref[...] = (
            jnp.dot(xyz, w0t_ref[0:3, :], preferred_element_type=jnp.float32)
            + lax.dot_general(feat_ref[...], w0t_ref[3:19, :], dn,
                              preferred_element_type=jnp.float32))
        tw1_ref[...] = (
            jnp.dot(xyz, w1t_ref[0:3, :], preferred_element_type=jnp.float32)
            + lax.dot_general(feat_ref[...], w1t_ref[3:19, :], dn,
                              preferred_element_type=jnp.float32))

    oh = (idx_ref[...] == perm_ref[...]).astype(jnp.float32)
    q = jnp.concatenate(
        [jnp.sum(oh * xtp_ref[d:d + 1, :], axis=1, keepdims=True)
         for d in range(3)], axis=1)
    nx_ref[...] = q

    qn = jnp.sum(q * q, axis=1, keepdims=True)
    pn = jnp.sum(xtp_ref[...] * xtp_ref[...], axis=0, keepdims=True)
    qp = jnp.dot(q, xtp_ref[...], preferred_element_type=jnp.float32)
    d2 = qn + pn - 2.0 * qp
    valid0 = d2 < np.float32(_R0 * _R0)
    valid1 = d2 < np.float32(_R1 * _R1)
    code_ref[...] = jnp.where(valid0, 3, jnp.where(valid1, 1, 0)
                              ).astype(jnp.int8)

    pack = jnp.where(valid0, np.float32(65537.0),
                     jnp.where(valid1, np.float32(1.0), np.float32(0.0)))
    chk = jnp.dot(pack, bd_ref[...],
                  preferred_element_type=jnp.float32)
    cum = _cumsum_lanes(chk.astype(jnp.int32))
    c0ch = jax.lax.shift_right_logical(cum, 16)
    c1ch = jnp.bitwise_and(cum, 65535)
    short = jnp.logical_or(c0ch < _NS0, c1ch < _NS1)
    need = jnp.sum(jnp.where(short, 1, 0), axis=1, keepdims=True) + 1
    stop_ref[...] = jnp.minimum(need, _WPR)


def _sc_body(words, stops, tw0, tw1, y0, y1,
             wbuf, sbuf, ib0, ib1, rv0, rv1, sem):
    wid = lax.axis_index("s") * _NC + lax.axis_index("c")
    row0 = wid * _RPW
    lanes = lax.iota(jnp.int32, 16)

    def batch_body(bi, carry):
        rowbase = row0 + bi * _RB
        pltpu.sync_copy(words.at[pl.ds(rowbase, _RB), :], wbuf)
        pltpu.sync_copy(stops.at[wid, pl.ds(bi * _RB, _RB)], sbuf)

        def zero16(i, c):
            ib1[pl.ds(i * 16, 16)] = jnp.zeros((16,), jnp.int32)
            return c
        lax.fori_loop(0, _RB * _NS1 // 16, zero16, 0)

        def zero0(i, c):
            ib0[pl.ds(i * 16, 16)] = jnp.zeros((16,), jnp.int32)
            return c
        lax.fori_loop(0, _RB * _NS0 // 16, zero0, 0)

        def row_body(r, c):
            rowg = rowbase + r
            b = lax.shift_right_logical(rowg, 10)
            bn = b * _N

            sv = sbuf[...]
            nv = jnp.max(jnp.where(lanes == r, sv, 0))

            def step(v, st):
                c0, c1 = st
                w = wbuf[r, pl.ds(v * 16, 16)]
                for t in range(4):
                    fld = jnp.bitwise_and(
                        lax.shift_right_logical(w, 8 * t), 255)
                    idxv = 64 * v + 16 * t + lanes
                    m1 = fld >= 1
                    h1 = jnp.where(m1, 1, 0)
                    pos1 = plsc.cumsum(h1) - 1 + c1
                    plsc.store_scatter(
                        ib1, [r * _NS1 + pos1], idxv,
                        mask=jnp.logical_and(m1, pos1 < _NS1))
                    c1 = c1 + jnp.sum(h1)
                    m0 = fld >= 2
                    h0 = jnp.where(m0, 1, 0)
                    pos0 = plsc.cumsum(h0) - 1 + c0
                    plsc.store_scatter(
                        ib0, [r * _NS0 + pos0], idxv,
                        mask=jnp.logical_and(m0, pos0 < _NS0))
                    c0 = c0 + jnp.sum(h0)
                return (c0, c1)

            c0, c1 = lax.fori_loop(0, nv, step, (0, 0))

            cur0 = ib0[pl.ds(r * _NS0, 16)]
            f0 = jnp.max(jnp.where(lanes == 0, cur0, 0))
            ib0[pl.ds(r * _NS0, 16)] = jnp.where(lanes < c0, cur0, f0) + bn
            cur1a = ib1[pl.ds(r * _NS1, 16)]
            f1 = jnp.max(jnp.where(lanes == 0, cur1a, 0))
            ib1[pl.ds(r * _NS1, 16)] = (
                jnp.where(lanes < c1, cur1a, f1) + bn)
            cur1b = ib1[pl.ds(r * _NS1 + 16, 16)]
            ib1[pl.ds(r * _NS1 + 16, 16)] = (
                jnp.where(lanes + 16 < c1, cur1b, f1) + bn)
            return c

        lax.fori_loop(0, _RB, row_body, 0)

        for ch in range(_RB * _NS0 // 128):
            pltpu.async_copy(tw0.at[ib0.at[pl.ds(ch * 128, 128)]],
                             rv0, sem).wait()
            pltpu.sync_copy(
                rv0, y0.at[pl.ds(rowbase * _NS0 + ch * 128, 128), :])
        for ch in range(_RB * _NS1 // 128):
            pltpu.async_copy(tw1.at[ib1.at[pl.ds(ch * 128, 128)]],
                             rv1, sem).wait()
            pltpu.sync_copy(
                rv1, y1.at[pl.ds(rowbase * _NS1 + ch * 128, 128), :])
        return carry

    lax.fori_loop(0, _NBATCH, batch_body, 0)


def _k3_body(y0r_ref, y1r_ref, nx_ref, w0t_ref, w1t_ref,
             y0_ref, y1_ref, st0_ref, st1_ref):
    b = pl.program_id(0)
    j = pl.program_id(1)

    @pl.when(jnp.logical_and(b == 0, j == 0))
    def _():
        st0_ref[...] = jnp.zeros_like(st0_ref)
        st1_ref[...] = jnp.zeros_like(st1_ref)

    q = nx_ref[...]

    def one_scale(yr_ref, wt_ref, y_ref, st_ref):
        center = jnp.dot(q, wt_ref[0:3, :],
                         preferred_element_type=jnp.float32)
        y3 = yr_ref[...] - center[:, None, :]
        y_ref[...] = y3
        t = jnp.sum(y3, axis=1)
        st_ref[0:1, :] += jnp.sum(t, axis=0)[None, :]
        t2 = jnp.sum(y3 * y3, axis=1)
        st_ref[1:2, :] += jnp.sum(t2, axis=0)[None, :]

    one_scale(y0r_ref, w0t_ref, y0_ref, st0_ref)
    one_scale(y1r_ref, w1t_ref, y1_ref, st1_ref)


def _k4_body(y0_ref, y1_ref, a0_ref, c0_ref, a1_ref, c1_ref,
             w0t_ref, w1t_ref,
             mx0_ref, mn0_ref, mx1_ref, mn1_ref, st0_ref, st1_ref):
    b = pl.program_id(0)
    j = pl.program_id(1)

    @pl.when(jnp.logical_and(b == 0, j == 0))
    def _():
        st0_ref[...] = jnp.zeros_like(st0_ref)
        st1_ref[...] = jnp.zeros_like(st1_ref)

    def one_scale(y_ref, a_ref, c_ref, wt_ref, mx_ref, mn_ref, st_ref, ns):
        y = y_ref[...]
        h = jnp.maximum(y * a_ref[...][:, None, :] + c_ref[...][:, None, :],
                        0.0)
        c1 = h.shape[2]
        wt = wt_ref[...]
        z = jnp.dot(h.reshape(_MB2 * ns, c1), wt,
                    preferred_element_type=jnp.float32)
        st_ref[0:1, :] += jnp.sum(z, axis=0)[None, :]
        st_ref[1:2, :] += jnp.sum(z * z, axis=0)[None, :]
        z3 = z.reshape(_MB2, ns, wt.shape[1])
        mx_ref[...] = jnp.max(z3, axis=1)
        mn_ref[...] = jnp.min(z3, axis=1)

    one_scale(y0_ref, a0_ref, c0_ref, w0t_ref, mx0_ref, mn0_ref, st0_ref, _NS0)
    one_scale(y1_ref, a1_ref, c1_ref, w1t_ref, mx1_ref, mn1_ref, st1_ref, _NS1)


def _k5_body(mx0_ref, mn0_ref, mx1_ref, mn1_ref,
             a0_ref, c0_ref, a1_ref, c1_ref, o_ref):
    def one_scale(mx_ref, mn_ref, a_ref, c_ref, lo, hi):
        a = a_ref[...]
        pooled = jnp.where(a >= 0.0, mx_ref[...], mn_ref[...])
        o = jnp.maximum(pooled * a + c_ref[...], 0.0)
        o_ref[lo:hi, :] = jnp.transpose(o, (1, 0))

    one_scale(mx0_ref, mn0_ref, a0_ref, c0_ref, 0, 32)
    one_scale(mx1_ref, mn1_ref, a1_ref, c1_ref, 32, 96)


def _finalize(st, cnt, g, bta):
    mean = st[0] / cnt
    var = st[1] / cnt - mean * mean
    scale = g / jnp.sqrt(var + _EPS)
    return (scale[None, :], (bta - mean * scale)[None, :])


def kernel(points_xyz, features, indices,
           w0_0, g0_0, b0_0, w0_1, g0_1, b0_1,
           w1_0, g1_0, b1_0, w1_1, g1_1, b1_1):
    f32 = jnp.float32
    xtp = jnp.transpose(points_xyz, (0, 2, 1))[:, :, _PERM]

    grid_a = (_B, _M // _MBA)
    bd = np.zeros((_N, 64), np.float32)
    bd[_JJ, _JJ // 64] = 1.0
    nx, code, stops, tw0, tw1 = pl.pallas_call(
        _k1_body,
        grid=grid_a,
        in_specs=[
            pl.BlockSpec((None, _MBA, 1), lambda b, j: (b, j, 0)),
            pl.BlockSpec((None, _N, 3), lambda b, j: (b, 0, 0)),
            pl.BlockSpec((None, _CIN, _N), lambda b, j: (b, 0, 0)),
            pl.BlockSpec((None, 3, _N), lambda b, j: (b, 0, 0)),
            pl.BlockSpec((1, _N), lambda b, j: (0, 0)),
            pl.BlockSpec((_N, 64), lambda b, j: (0, 0)),
            pl.BlockSpec((19, 16), lambda b, j: (0, 0)),
            pl.BlockSpec((19, 32), lambda b, j: (0, 0)),
        ],
        out_specs=[
            pl.BlockSpec((None, _MBA, 3), lambda b, j: (b, j, 0)),
            pl.BlockSpec((None, _MBA, _N), lambda b, j: (b, j, 0)),
            pl.BlockSpec((None, _MBA, 1), lambda b, j: (b, j, 0)),
            pl.BlockSpec((None, _N, 16), lambda b, j: (b, 0, 0)),
            pl.BlockSpec((None, _N, 32), lambda b, j: (b, 0, 0)),
        ],
        out_shape=[
            jax.ShapeDtypeStruct((_B, _M, 3), f32),
            jax.ShapeDtypeStruct((_B, _M, _N), jnp.int8),
            jax.ShapeDtypeStruct((_B, _M, 1), jnp.int32),
            jax.ShapeDtypeStruct((_B, _N, 16), f32),
            jax.ShapeDtypeStruct((_B, _N, 32), f32),
        ],
    )(indices.reshape(_B, _M, 1), points_xyz, features, xtp,
      jnp.asarray(_PERM.astype(np.int32)[None, :]),
      jnp.asarray(bd), w0_0.T, w1_0.T)

    words = lax.bitcast_convert_type(
        code.reshape(_B * _M, _N // 4, 4), jnp.int32)

    mesh = plsc.VectorSubcoreMesh(core_axis_name="c", subcore_axis_name="s")
    y0r, y1r = pl.kernel(
        _sc_body,
        mesh=mesh,
        compiler_params=pltpu.CompilerParams(needs_layout_passes=False,
                                             use_tc_tiling_on_sc=False),
        out_type=[
            jax.ShapeDtypeStruct((_B * _M * _NS0, 16), f32),
            jax.ShapeDtypeStruct((_B * _M * _NS1, 32), f32),
        ],
        scratch_types=[
            pltpu.VMEM((_RB, _N // 4), jnp.int32),
            pltpu.VMEM((_RB,), jnp.int32),
            pltpu.VMEM((_RB * _NS0,), jnp.int32),
            pltpu.VMEM((_RB * _NS1,), jnp.int32),
            pltpu.VMEM((128, 16), f32),
            pltpu.VMEM((128, 32), f32),
            pltpu.SemaphoreType.DMA,
        ],
    )(words, stops.reshape(_NW, _RPW),
      tw0.reshape(_B * _N, 16), tw1.reshape(_B * _N, 32))

    y0r = y0r.reshape(_B, _M, _NS0, 16)
    y1r = y1r.reshape(_B, _M, _NS1, 32)

    grid_3 = (_B, _M // _MB3)
    y0, y1, st0, st1 = pl.pallas_call(
        _k3_body,
        grid=grid_3,
        in_specs=[
            pl.BlockSpec((None, _MB3, _NS0, 16), lambda b, j: (b, j, 0, 0)),
            pl.BlockSpec((None, _MB3, _NS1, 32), lambda b, j: (b, j, 0, 0)),
            pl.BlockSpec((None, _MB3, 3), lambda b, j: (b, j, 0)),
            pl.BlockSpec((19, 16), lambda b, j: (0, 0)),
            pl.BlockSpec((19, 32), lambda b, j: (0, 0)),
        ],
        out_specs=[
            pl.BlockSpec((None, _MB3, _NS0, 16), lambda b, j: (b, j, 0, 0)),
            pl.BlockSpec((None, _MB3, _NS1, 32), lambda b, j: (b, j, 0, 0)),
            pl.BlockSpec((2, 16), lambda b, j: (0, 0)),
            pl.BlockSpec((2, 32), lambda b, j: (0, 0)),
        ],
        out_shape=[
            jax.ShapeDtypeStruct((_B, _M, _NS0, 16), f32),
            jax.ShapeDtypeStruct((_B, _M, _NS1, 32), f32),
            jax.ShapeDtypeStruct((2, 16), f32),
            jax.ShapeDtypeStruct((2, 32), f32),
        ],
    )(y0r, y1r, nx, w0_0.T, w1_0.T)

    cnt0 = np.float32(_B * _M * _NS0)
    cnt1 = np.float32(_B * _M * _NS1)
    a0, c0 = _finalize(st0, cnt0, g0_0, b0_0)
    a1, c1 = _finalize(st1, cnt1, g1_0, b1_0)

    grid_b = (_B, _M // _MB2)
    mx0, mn0, mx1, mn1, st20, st21 = pl.pallas_call(
        _k4_body,
        grid=grid_b,
        in_specs=[
            pl.BlockSpec((None, _MB2, _NS0, 16), lambda b, j: (b, j, 0, 0)),
            pl.BlockSpec((None, _MB2, _NS1, 32), lambda b, j: (b, j, 0, 0)),
            pl.BlockSpec((1, 16), lambda b, j: (0, 0)),
            pl.BlockSpec((1, 16), lambda b, j: (0, 0)),
            pl.BlockSpec((1, 32), lambda b, j: (0, 0)),
            pl.BlockSpec((1, 32), lambda b, j: (0, 0)),
            pl.BlockSpec((16, 32), lambda b, j: (0, 0)),
            pl.BlockSpec((32, 64), lambda b, j: (0, 0)),
        ],
        out_specs=[
            pl.BlockSpec((None, _MB2, 32), lambda b, j: (b, j, 0)),
            pl.BlockSpec((None, _MB2, 32), lambda b, j: (b, j, 0)),
            pl.BlockSpec((None, _MB2, 64), lambda b, j: (b, j, 0)),
            pl.BlockSpec((None, _MB2, 64), lambda b, j: (b, j, 0)),
            pl.BlockSpec((2, 32), lambda b, j: (0, 0)),
            pl.BlockSpec((2, 64), lambda b, j: (0, 0)),
        ],
        out_shape=[
            jax.ShapeDtypeStruct((_B, _M, 32), f32),
            jax.ShapeDtypeStruct((_B, _M, 32), f32),
            jax.ShapeDtypeStruct((_B, _M, 64), f32),
            jax.ShapeDtypeStruct((_B, _M, 64), f32),
            jax.ShapeDtypeStruct((2, 32), f32),
            jax.ShapeDtypeStruct((2, 64), f32),
        ],
    )(y0, y1, a0, c0, a1, c1, w0_1.T, w1_1.T)

    a20, c20 = _finalize(st20, cnt0, g0_1, b0_1)
    a21, c21 = _finalize(st21, cnt1, g1_1, b1_1)

    new_features = pl.pallas_call(
        _k5_body,
        grid=(_B,),
        in_specs=[
            pl.BlockSpec((None, _M, 32), lambda b: (b, 0, 0)),
            pl.BlockSpec((None, _M, 32), lambda b: (b, 0, 0)),
            pl.BlockSpec((None, _M, 64), lambda b: (b, 0, 0)),
            pl.BlockSpec((None, _M, 64), lambda b: (b, 0, 0)),
            pl.BlockSpec((1, 32), lambda b: (0, 0)),
            pl.BlockSpec((1, 32), lambda b: (0, 0)),
            pl.BlockSpec((1, 64), lambda b: (0, 0)),
            pl.BlockSpec((1, 64), lambda b: (0, 0)),
        ],
        out_specs=pl.BlockSpec((None, 96, _M), lambda b: (b, 0, 0)),
        out_shape=jax.ShapeDtypeStruct((_B, 96, _M), f32),
    )(mx0, mn0, mx1, mn1, a20, c20, a21, c21)

    return (nx, new_features, indices)


# speedup vs baseline: 1.0840x; 1.0840x over previous
import functools

import jax
import jax.numpy as jnp
import numpy as np
from jax import lax
from jax.experimental import pallas as pl
from jax.experimental.pallas import tpu as pltpu
from jax.experimental.pallas import tpu_sc as plsc

_B, _N, _CIN = 8, 4096, 16
_M = 1024
_R0, _R1 = 0.2, 0.4
_NS0, _NS1 = 16, 32
_EPS = 1e-5
_MBA = 128
_MB3 = 128
_MB2 = 128

_NC, _NSC = 2, 16
_NW = _NC * _NSC
_RPW = (_B * _M) // _NW
_RB = 16
_NBATCH = _RPW // _RB
_WPR = _N // 4 // 16

_JJ = np.arange(_N)
_PERM = (_JJ // 64) * 64 + 16 * (_JJ % 4) + (_JJ % 64) // 4


def _cumsum_lanes(x):
    lane = jax.lax.broadcasted_iota(jnp.int32, x.shape, 1)
    s = 1
    while s < x.shape[1]:
        shifted = pltpu.roll(x, s, 1)
        x = x + jnp.where(lane >= s, shifted, 0)
        s *= 2
    return x


def _k1_body(idx_ref, xyz_ref, feat_ref, xtp_ref, perm_ref, bd_ref,
             w0t_ref, w1t_ref,
             nx_ref, code_ref, stop_ref, tw0_ref, tw1_ref):
    j = pl.program_id(1)
    xyz = xyz_ref[...]

    @pl.when(j == 0)
    def _():
        dn = (((0,), (0,)), ((), ()))
        tw0_ref[...] = (
            jnp.dot(xyz, w0t_ref[0:3, :], preferred_element_type=jnp.float32)
            + lax.dot_general(feat_ref[...], w0t_ref[3:19, :], dn,
                              preferred_element_type=jnp.float32))
        tw1_ref[...] = (
            jnp.dot(xyz, w1t_ref[0:3, :], preferred_element_type=jnp.float32)
            + lax.dot_general(feat_ref[...], w1t_ref[3:19, :], dn,
                              preferred_element_type=jnp.float32))

    oh = (idx_ref[...] == perm_ref[...]).astype(jnp.float32)
    q = jnp.concatenate(
        [jnp.sum(oh * xtp_ref[d:d + 1, :], axis=1, keepdims=True)
         for d in range(3)], axis=1)
    nx_ref[...] = q

    qn = jnp.sum(q * q, axis=1, keepdims=True)
    pn = jnp.sum(xtp_ref[...] * xtp_ref[...], axis=0, keepdims=True)
    qp = jnp.dot(q, xtp_ref[...], preferred_element_type=jnp.float32)
    d2 = qn + pn - 2.0 * qp
    valid0 = d2 < np.float32(_R0 * _R0)
    valid1 = d2 < np.float32(_R1 * _R1)
    code_ref[...] = jnp.where(valid0, 3, jnp.where(valid1, 1, 0)
                              ).astype(jnp.int8)

    pack = jnp.where(valid0, np.float32(65537.0),
                     jnp.where(valid1, np.float32(1.0), np.float32(0.0)))
    chk = jnp.dot(pack, bd_ref[...],
                  preferred_element_type=jnp.float32)
    cum = _cumsum_lanes(chk.astype(jnp.int32))
    c0ch = jax.lax.shift_right_logical(cum, 16)
    c1ch = jnp.bitwise_and(cum, 65535)
    short = jnp.logical_or(c0ch < _NS0, c1ch < _NS1)
    need = jnp.sum(jnp.where(short, 1, 0), axis=1, keepdims=True) + 1
    stop_ref[...] = jnp.minimum(need, _WPR)


def _sc_body(words, stops, tw0, tw1, y0, y1,
             wbuf, sbuf, ib0, ib1, rv0, rv1, sem):
    wid = lax.axis_index("s") * _NC + lax.axis_index("c")
    row0 = wid * _RPW
    lanes = lax.iota(jnp.int32, 16)

    def batch_body(bi, carry):
        rowbase = row0 + bi * _RB
        pltpu.sync_copy(words.at[pl.ds(rowbase, _RB), :], wbuf)
        pltpu.sync_copy(stops.at[wid, pl.ds(bi * _RB, _RB)], sbuf)

        def zero16(i, c):
            ib1[pl.ds(i * 16, 16)] = jnp.zeros((16,), jnp.int32)
            return c
        lax.fori_loop(0, _RB * _NS1 // 16, zero16, 0)

        def zero0(i, c):
            ib0[pl.ds(i * 16, 16)] = jnp.zeros((16,), jnp.int32)
            return c
        lax.fori_loop(0, _RB * _NS0 // 16, zero0, 0)

        def row_body(r, c):
            rowg = rowbase + r
            b = lax.shift_right_logical(rowg, 10)
            bn = b * _N

            sv = sbuf[...]
            nv = jnp.max(jnp.where(lanes == r, sv, 0))

            def step(v, st):
                c0, c1 = st
                w = wbuf[r, pl.ds(v * 16, 16)]
                for t in range(4):
                    fld = jnp.bitwise_and(
                        lax.shift_right_logical(w, 8 * t), 255)
                    idxv = 64 * v + 16 * t + lanes
                    m1 = fld >= 1
                    h1 = jnp.where(m1, 1, 0)
                    pos1 = plsc.cumsum(h1) - 1 + c1
                    plsc.store_scatter(
                        ib1, [r * _NS1 + pos1], idxv,
                        mask=jnp.logical_and(m1, pos1 < _NS1))
                    c1 = c1 + jnp.sum(h1)
                    m0 = fld >= 2
                    h0 = jnp.where(m0, 1, 0)
                    pos0 = plsc.cumsum(h0) - 1 + c0
                    plsc.store_scatter(
                        ib0, [r * _NS0 + pos0], idxv,
                        mask=jnp.logical_and(m0, pos0 < _NS0))
                    c0 = c0 + jnp.sum(h0)
                return (c0, c1)

            c0, c1 = lax.fori_loop(0, nv, step, (0, 0))

            cur0 = ib0[pl.ds(r * _NS0, 16)]
            f0 = jnp.max(jnp.where(lanes == 0, cur0, 0))
            ib0[pl.ds(r * _NS0, 16)] = jnp.where(lanes < c0, cur0, f0) + bn
            cur1a = ib1[pl.ds(r * _NS1, 16)]
            f1 = jnp.max(jnp.where(lanes == 0, cur1a, 0))
            ib1[pl.ds(r * _NS1, 16)] = (
                jnp.where(lanes < c1, cur1a, f1) + bn)
            cur1b = ib1[pl.ds(r * _NS1 + 16, 16)]
            ib1[pl.ds(r * _NS1 + 16, 16)] = (
                jnp.where(lanes + 16 < c1, cur1b, f1) + bn)
            return c

        lax.fori_loop(0, _RB, row_body, 0)

        for ch in range(_RB * _NS0 // 128):
            pltpu.async_copy(tw0.at[ib0.at[pl.ds(ch * 128, 128)]],
                             rv0, sem).wait()
            pltpu.sync_copy(
                rv0, y0.at[pl.ds(rowbase * _NS0 + ch * 128, 128), :])
        for ch in range(_RB * _NS1 // 128):
            pltpu.async_copy(tw1.at[ib1.at[pl.ds(ch * 128, 128)]],
                             rv1, sem).wait()
            pltpu.sync_copy(
                rv1, y1.at[pl.ds(rowbase * _NS1 + ch * 128, 128), :])
        return carry

    lax.fori_loop(0, _NBATCH, batch_body, 0)


def _k3_body(y0r_ref, y1r_ref, nx_ref, w0t_ref, w1t_ref,
             y0_ref, y1_ref, st0_ref, st1_ref):
    b = pl.program_id(0)
    j = pl.program_id(1)

    @pl.when(jnp.logical_and(b == 0, j == 0))
    def _():
        st0_ref[...] = jnp.zeros_like(st0_ref)
        st1_ref[...] = jnp.zeros_like(st1_ref)

    q = nx_ref[...]

    def one_scale(yr_ref, wt_ref, y_ref, st_ref):
        center = jnp.dot(q, wt_ref[0:3, :],
                         preferred_element_type=jnp.float32)
        y3 = yr_ref[...] - center[:, None, :]
        y_ref[...] = y3
        t = jnp.sum(y3, axis=1)
        st_ref[0:1, :] += jnp.sum(t, axis=0)[None, :]
        t2 = jnp.sum(y3 * y3, axis=1)
        st_ref[1:2, :] += jnp.sum(t2, axis=0)[None, :]

    one_scale(y0r_ref, w0t_ref, y0_ref, st0_ref)
    one_scale(y1r_ref, w1t_ref, y1_ref, st1_ref)


def _k4_body(y0_ref, y1_ref, a0_ref, c0_ref, a1_ref, c1_ref,
             w0t_ref, w1t_ref,
             mx0_ref, mn0_ref, mx1_ref, mn1_ref, st0_ref, st1_ref):
    b = pl.program_id(0)
    j = pl.program_id(1)

    @pl.when(jnp.logical_and(b == 0, j == 0))
    def _():
        st0_ref[...] = jnp.zeros_like(st0_ref)
        st1_ref[...] = jnp.zeros_like(st1_ref)

    def one_scale(y_ref, a_ref, c_ref, wt_ref, mx_ref, mn_ref, st_ref, ns):
        y = y_ref[...]
        h = jnp.maximum(y * a_ref[...][:, None, :] + c_ref[...][:, None, :],
                        0.0)
        c1 = h.shape[2]
        wt = wt_ref[...]
        z = jnp.dot(h.reshape(_MB2 * ns, c1), wt,
                    preferred_element_type=jnp.float32)
        st_ref[0:1, :] += jnp.sum(z, axis=0)[None, :]
        st_ref[1:2, :] += jnp.sum(z * z, axis=0)[None, :]
        z3 = z.reshape(_MB2, ns, wt.shape[1])
        mx_ref[...] = jnp.max(z3, axis=1)
        mn_ref[...] = jnp.min(z3, axis=1)

    one_scale(y0_ref, a0_ref, c0_ref, w0t_ref, mx0_ref, mn0_ref, st0_ref, _NS0)
    one_scale(y1_ref, a1_ref, c1_ref, w1t_ref, mx1_ref, mn1_ref, st1_ref, _NS1)


def _k5_body(mx0_ref, mn0_ref, mx1_ref, mn1_ref,
             a0_ref, c0_ref, a1_ref, c1_ref, o_ref):
    def one_scale(mx_ref, mn_ref, a_ref, c_ref, lo, hi):
        a = a_ref[...]
        pooled = jnp.where(a >= 0.0, mx_ref[...], mn_ref[...])
        o = jnp.maximum(pooled * a + c_ref[...], 0.0)
        o_ref[lo:hi, :] = jnp.transpose(o, (1, 0))

    one_scale(mx0_ref, mn0_ref, a0_ref, c0_ref, 0, 32)
    one_scale(mx1_ref, mn1_ref, a1_ref, c1_ref, 32, 96)


def _finalize(st, cnt, g, bta):
    mean = st[0] / cnt
    var = st[1] / cnt - mean * mean
    scale = g / jnp.sqrt(var + _EPS)
    return (scale[None, :], (bta - mean * scale)[None, :])


def kernel(points_xyz, features, indices,
           w0_0, g0_0, b0_0, w0_1, g0_1, b0_1,
           w1_0, g1_0, b1_0, w1_1, g1_1, b1_1):
    f32 = jnp.float32
    xtp = jnp.transpose(points_xyz, (0, 2, 1))[:, :, _PERM]

    grid_a = (_B, _M // _MBA)
    bd = np.zeros((_N, 64), np.float32)
    bd[_JJ, _JJ // 64] = 1.0
    nx, code, stops, tw0, tw1 = pl.pallas_call(
        _k1_body,
        grid=grid_a,
        in_specs=[
            pl.BlockSpec((None, _MBA, 1), lambda b, j: (b, j, 0)),
            pl.BlockSpec((None, _N, 3), lambda b, j: (b, 0, 0)),
            pl.BlockSpec((None, _CIN, _N), lambda b, j: (b, 0, 0)),
            pl.BlockSpec((None, 3, _N), lambda b, j: (b, 0, 0)),
            pl.BlockSpec((1, _N), lambda b, j: (0, 0)),
            pl.BlockSpec((_N, 64), lambda b, j: (0, 0)),
            pl.BlockSpec((19, 16), lambda b, j: (0, 0)),
            pl.BlockSpec((19, 32), lambda b, j: (0, 0)),
        ],
        out_specs=[
            pl.BlockSpec((None, _MBA, 3), lambda b, j: (b, j, 0)),
            pl.BlockSpec((None, _MBA, _N), lambda b, j: (b, j, 0)),
            pl.BlockSpec((None, _MBA, 1), lambda b, j: (b, j, 0)),
            pl.BlockSpec((None, _N, 16), lambda b, j: (b, 0, 0)),
            pl.BlockSpec((None, _N, 32), lambda b, j: (b, 0, 0)),
        ],
        out_shape=[
            jax.ShapeDtypeStruct((_B, _M, 3), f32),
            jax.ShapeDtypeStruct((_B, _M, _N), jnp.int8),
            jax.ShapeDtypeStruct((_B, _M, 1), jnp.int32),
            jax.ShapeDtypeStruct((_B, _N, 16), f32),
            jax.ShapeDtypeStruct((_B, _N, 32), f32),
        ],
    )(indices.reshape(_B, _M, 1), points_xyz, features, xtp,
      jnp.asarray(_PERM.astype(np.int32)[None, :]),
      jnp.asarray(bd), w0_0.T, w1_0.T)

    words = lax.bitcast_convert_type(
        code.reshape(_B * _M, _N // 4, 4), jnp.int32)

    mesh = plsc.VectorSubcoreMesh(core_axis_name="c", subcore_axis_name="s")
    y0r, y1r = pl.kernel(
        _sc_body,
        mesh=mesh,
        compiler_params=pltpu.CompilerParams(needs_layout_passes=False,
                                             use_tc_tiling_on_sc=False),
        out_type=[
            jax.ShapeDtypeStruct((_B * _M * _NS0, 16), f32),
            jax.ShapeDtypeStruct((_B * _M * _NS1, 32), f32),
        ],
        scratch_types=[
            pltpu.VMEM((_RB, _N // 4), jnp.int32),
            pltpu.VMEM((_RB,), jnp.int32),
            pltpu.VMEM((_RB * _NS0,), jnp.int32),
            pltpu.VMEM((_RB * _NS1,), jnp.int32),
            pltpu.VMEM((128, 16), f32),
            pltpu.VMEM((128, 32), f32),
            pltpu.SemaphoreType.DMA,
        ],
    )(words, stops.reshape(_NW, _RPW),
      tw0.reshape(_B * _N, 16), tw1.reshape(_B * _N, 32))

    y0r = y0r.reshape(_B, _M, _NS0, 16)
    y1r = y1r.reshape(_B, _M, _NS1, 32)

    grid_3 = (_B, _M // _MB3)
    y0, y1, st0, st1 = pl.pallas_call(
        _k3_body,
        grid=grid_3,
        in_specs=[
            pl.BlockSpec((None, _MB3, _NS0, 16), lambda b, j: (b, j, 0, 0)),
            pl.BlockSpec((None, _MB3, _NS1, 32), lambda b, j: (b, j, 0, 0)),
            pl.BlockSpec((None, _MB3, 3), lambda b, j: (b, j, 0)),
            pl.BlockSpec((19, 16), lambda b, j: (0, 0)),
            pl.BlockSpec((19, 32), lambda b, j: (0, 0)),
        ],
        out_specs=[
            pl.BlockSpec((None, _MB3, _NS0, 16), lambda b, j: (b, j, 0, 0)),
            pl.BlockSpec((None, _MB3, _NS1, 32), lambda b, j: (b, j, 0, 0)),
            pl.BlockSpec((2, 16), lambda b, j: (0, 0)),
            pl.BlockSpec((2, 32), lambda b, j: (0, 0)),
        ],
        out_shape=[
            jax.ShapeDtypeStruct((_B, _M, _NS0, 16), f32),
            jax.ShapeDtypeStruct((_B, _M, _NS1, 32), f32),
            jax.ShapeDtypeStruct((2, 16), f32),
            jax.ShapeDtypeStruct((2, 32), f32),
        ],
    )(y0r, y1r, nx, w0_0.T, w1_0.T)

    cnt0 = np.float32(_B * _M * _NS0)
    cnt1 = np.float32(_B * _M * _NS1)
    a0, c0 = _finalize(st0, cnt0, g0_0, b0_0)
    a1, c1 = _finalize(st1, cnt1, g1_0, b1_0)

    grid_b = (_B, _M // _MB2)
    mx0, mn0, mx1, mn1, st20, st21 = pl.pallas_call(
        _k4_body,
        grid=grid_b,
        in_specs=[
            pl.BlockSpec((None, _MB2, _NS0, 16), lambda b, j: (b, j, 0, 0)),
            pl.BlockSpec((None, _MB2, _NS1, 32), lambda b, j: (b, j, 0, 0)),
            pl.BlockSpec((1, 16), lambda b, j: (0, 0)),
            pl.BlockSpec((1, 16), lambda b, j: (0, 0)),
            pl.BlockSpec((1, 32), lambda b, j: (0, 0)),
            pl.BlockSpec((1, 32), lambda b, j: (0, 0)),
            pl.BlockSpec((16, 32), lambda b, j: (0, 0)),
            pl.BlockSpec((32, 64), lambda b, j: (0, 0)),
        ],
        out_specs=[
            pl.BlockSpec((None, _MB2, 32), lambda b, j: (b, j, 0)),
            pl.BlockSpec((None, _MB2, 32), lambda b, j: (b, j, 0)),
            pl.BlockSpec((None, _MB2, 64), lambda b, j: (b, j, 0)),
            pl.BlockSpec((None, _MB2, 64), lambda b, j: (b, j, 0)),
            pl.BlockSpec((2, 32), lambda b, j: (0, 0)),
            pl.BlockSpec((2, 64), lambda b, j: (0, 0)),
        ],
        out_shape=[
            jax.ShapeDtypeStruct((_B, _M, 32), f32),
            jax.ShapeDtypeStruct((_B, _M, 32), f32),
            jax.ShapeDtypeStruct((_B, _M, 64), f32),
            jax.ShapeDtypeStruct((_B, _M, 64), f32),
            jax.ShapeDtypeStruct((2, 32), f32),
            jax.ShapeDtypeStruct((2, 64), f32),
        ],
    )(y0, y1, a0, c0, a1, c1, w0_1.T, w1_1.T)

    a20, c20 = _finalize(st20, cnt0, g0_1, b0_1)
    a21, c21 = _finalize(st21, cnt1, g1_1, b1_1)

    new_features = pl.pallas_call(
        _k5_body,
        grid=(_B,),
        in_specs=[
            pl.BlockSpec((None, _M, 32), lambda b: (b, 0, 0)),
            pl.BlockSpec((None, _M, 32), lambda b: (b, 0, 0)),
            pl.BlockSpec((None, _M, 64), lambda b: (b, 0, 0)),
            pl.BlockSpec((None, _M, 64), lambda b: (b, 0, 0)),
            pl.BlockSpec((1, 32), lambda b: (0, 0)),
            pl.BlockSpec((1, 32), lambda b: (0, 0)),
            pl.BlockSpec((1, 64), lambda b: (0, 0)),
            pl.BlockSpec((1, 64), lambda b: (0, 0)),
        ],
        out_specs=pl.BlockSpec((None, 96, _M), lambda b: (b, 0, 0)),
        out_shape=jax.ShapeDtypeStruct((_B, 96, _M), f32),
    )(mx0, mn0, mx1, mn1, a20, c20, a21, c21)

    return (nx, new_features, indices)


# speedup vs baseline: 1.1410x; 1.0526x over previous
import functools

import jax
import jax.numpy as jnp
import numpy as np
from jax import lax
from jax.experimental import pallas as pl
from jax.experimental.pallas import tpu as pltpu
from jax.experimental.pallas import tpu_sc as plsc

_B, _N, _CIN = 8, 4096, 16
_M = 1024
_R0, _R1 = 0.2, 0.4
_NS0, _NS1 = 16, 32
_EPS = 1e-5
_MBA = 256
_MB3 = 128
_MB2 = 128

_NC, _NSC = 2, 16
_NW = _NC * _NSC
_RPW = (_B * _M) // _NW
_RB = 16
_NBATCH = _RPW // _RB
_WPR = _N // 4 // 16

_JJ = np.arange(_N)
_PERM = (_JJ // 64) * 64 + 16 * (_JJ % 4) + (_JJ % 64) // 4


def _cumsum_lanes(x):
    lane = jax.lax.broadcasted_iota(jnp.int32, x.shape, 1)
    s = 1
    while s < x.shape[1]:
        shifted = pltpu.roll(x, s, 1)
        x = x + jnp.where(lane >= s, shifted, 0)
        s *= 2
    return x


def _k1_body(idx_ref, xyz_ref, feat_ref, xtp_ref, perm_ref, bd_ref,
             w0t_ref, w1t_ref,
             nx_ref, code_ref, stop_ref, tw0_ref, tw1_ref):
    j = pl.program_id(1)
    xyz = xyz_ref[...]

    @pl.when(j == 0)
    def _():
        dn = (((0,), (0,)), ((), ()))
        tw0_ref[...] = (
            jnp.dot(xyz, w0t_ref[0:3, :], preferred_element_type=jnp.float32)
            + lax.dot_general(feat_ref[...], w0t_ref[3:19, :], dn,
                              preferred_element_type=jnp.float32))
        tw1_ref[...] = (
            jnp.dot(xyz, w1t_ref[0:3, :], preferred_element_type=jnp.float32)
            + lax.dot_general(feat_ref[...], w1t_ref[3:19, :], dn,
                              preferred_element_type=jnp.float32))

    oh = (idx_ref[...] == perm_ref[...]).astype(jnp.float32)
    q = jnp.concatenate(
        [jnp.sum(oh * xtp_ref[d:d + 1, :], axis=1, keepdims=True)
         for d in range(3)], axis=1)
    nx_ref[...] = q

    qn = jnp.sum(q * q, axis=1, keepdims=True)
    pn = jnp.sum(xtp_ref[...] * xtp_ref[...], axis=0, keepdims=True)
    qp = jnp.dot(q, xtp_ref[...], preferred_element_type=jnp.float32)
    d2 = qn + pn - 2.0 * qp
    valid0 = d2 < np.float32(_R0 * _R0)
    valid1 = d2 < np.float32(_R1 * _R1)
    code_ref[...] = jnp.where(valid0, 3, jnp.where(valid1, 1, 0)
                              ).astype(jnp.int8)

    pack = jnp.where(valid0, np.float32(65537.0),
                     jnp.where(valid1, np.float32(1.0), np.float32(0.0)))
    chk = jnp.dot(pack, bd_ref[...],
                  preferred_element_type=jnp.float32)
    cum = _cumsum_lanes(chk.astype(jnp.int32))
    c0ch = jax.lax.shift_right_logical(cum, 16)
    c1ch = jnp.bitwise_and(cum, 65535)
    short = jnp.logical_or(c0ch < _NS0, c1ch < _NS1)
    need = jnp.sum(jnp.where(short, 1, 0), axis=1, keepdims=True) + 1
    stop_ref[...] = jnp.minimum(need, _WPR)


def _sc_body(words, stops, tw0, tw1, y0, y1,
             wbuf, sbuf, ib0, ib1, rv0, rv1, sem):
    wid = lax.axis_index("s") * _NC + lax.axis_index("c")
    row0 = wid * _RPW
    lanes = lax.iota(jnp.int32, 16)

    def batch_body(bi, carry):
        rowbase = row0 + bi * _RB
        pltpu.sync_copy(words.at[pl.ds(rowbase, _RB), :], wbuf)
        pltpu.sync_copy(stops.at[wid, pl.ds(bi * _RB, _RB)], sbuf)

        def zero16(i, c):
            ib1[pl.ds(i * 16, 16)] = jnp.zeros((16,), jnp.int32)
            return c
        lax.fori_loop(0, _RB * _NS1 // 16, zero16, 0)

        def zero0(i, c):
            ib0[pl.ds(i * 16, 16)] = jnp.zeros((16,), jnp.int32)
            return c
        lax.fori_loop(0, _RB * _NS0 // 16, zero0, 0)

        def row_body(r, c):
            rowg = rowbase + r
            b = lax.shift_right_logical(rowg, 10)
            bn = b * _N

            sv = sbuf[...]
            nv = jnp.max(jnp.where(lanes == r, sv, 0))

            def step(v, st):
                c0, c1 = st
                w = wbuf[r, pl.ds(v * 16, 16)]
                for t in range(4):
                    fld = jnp.bitwise_and(
                        lax.shift_right_logical(w, 8 * t), 255)
                    idxv = 64 * v + 16 * t + lanes
                    m1 = fld >= 1
                    h1 = jnp.where(m1, 1, 0)
                    pos1 = plsc.cumsum(h1) - 1 + c1
                    plsc.store_scatter(
                        ib1, [r * _NS1 + pos1], idxv,
                        mask=jnp.logical_and(m1, pos1 < _NS1))
                    c1 = c1 + jnp.sum(h1)
                    m0 = fld >= 2
                    h0 = jnp.where(m0, 1, 0)
                    pos0 = plsc.cumsum(h0) - 1 + c0
                    plsc.store_scatter(
                        ib0, [r * _NS0 + pos0], idxv,
                        mask=jnp.logical_and(m0, pos0 < _NS0))
                    c0 = c0 + jnp.sum(h0)
                return (c0, c1)

            c0, c1 = lax.fori_loop(0, nv, step, (0, 0))

            cur0 = ib0[pl.ds(r * _NS0, 16)]
            f0 = jnp.max(jnp.where(lanes == 0, cur0, 0))
            ib0[pl.ds(r * _NS0, 16)] = jnp.where(lanes < c0, cur0, f0) + bn
            cur1a = ib1[pl.ds(r * _NS1, 16)]
            f1 = jnp.max(jnp.where(lanes == 0, cur1a, 0))
            ib1[pl.ds(r * _NS1, 16)] = (
                jnp.where(lanes < c1, cur1a, f1) + bn)
            cur1b = ib1[pl.ds(r * _NS1 + 16, 16)]
            ib1[pl.ds(r * _NS1 + 16, 16)] = (
                jnp.where(lanes + 16 < c1, cur1b, f1) + bn)
            return c

        lax.fori_loop(0, _RB, row_body, 0)

        for ch in range(_RB * _NS0 // 128):
            pltpu.async_copy(tw0.at[ib0.at[pl.ds(ch * 128, 128)]],
                             rv0, sem).wait()
            pltpu.sync_copy(
                rv0, y0.at[pl.ds(rowbase * _NS0 + ch * 128, 128), :])
        for ch in range(_RB * _NS1 // 128):
            pltpu.async_copy(tw1.at[ib1.at[pl.ds(ch * 128, 128)]],
                             rv1, sem).wait()
            pltpu.sync_copy(
                rv1, y1.at[pl.ds(rowbase * _NS1 + ch * 128, 128), :])
        return carry

    lax.fori_loop(0, _NBATCH, batch_body, 0)


def _k3_body(y0r_ref, y1r_ref, nx_ref, w0t_ref, w1t_ref,
             y0_ref, y1_ref, st0_ref, st1_ref):
    b = pl.program_id(0)
    j = pl.program_id(1)

    @pl.when(jnp.logical_and(b == 0, j == 0))
    def _():
        st0_ref[...] = jnp.zeros_like(st0_ref)
        st1_ref[...] = jnp.zeros_like(st1_ref)

    q = nx_ref[...]

    def one_scale(yr_ref, wt_ref, y_ref, st_ref):
        center = jnp.dot(q, wt_ref[0:3, :],
                         preferred_element_type=jnp.float32)
        y3 = yr_ref[...] - center[:, None, :]
        y_ref[...] = y3
        t = jnp.sum(y3, axis=1)
        st_ref[0:1, :] += jnp.sum(t, axis=0)[None, :]
        t2 = jnp.sum(y3 * y3, axis=1)
        st_ref[1:2, :] += jnp.sum(t2, axis=0)[None, :]

    one_scale(y0r_ref, w0t_ref, y0_ref, st0_ref)
    one_scale(y1r_ref, w1t_ref, y1_ref, st1_ref)


def _k4_body(y0_ref, y1_ref, a0_ref, c0_ref, a1_ref, c1_ref,
             w0t_ref, w1t_ref,
             mx0_ref, mn0_ref, mx1_ref, mn1_ref, st0_ref, st1_ref):
    b = pl.program_id(0)
    j = pl.program_id(1)

    @pl.when(jnp.logical_and(b == 0, j == 0))
    def _():
        st0_ref[...] = jnp.zeros_like(st0_ref)
        st1_ref[...] = jnp.zeros_like(st1_ref)

    def one_scale(y_ref, a_ref, c_ref, wt_ref, mx_ref, mn_ref, st_ref, ns):
        y = y_ref[...]
        h = jnp.maximum(y * a_ref[...][:, None, :] + c_ref[...][:, None, :],
                        0.0)
        c1 = h.shape[2]
        wt = wt_ref[...]
        z = jnp.dot(h.reshape(_MB2 * ns, c1), wt,
                    preferred_element_type=jnp.float32)
        st_ref[0:1, :] += jnp.sum(z, axis=0)[None, :]
        st_ref[1:2, :] += jnp.sum(z * z, axis=0)[None, :]
        z3 = z.reshape(_MB2, ns, wt.shape[1])
        mx_ref[...] = jnp.max(z3, axis=1)
        mn_ref[...] = jnp.min(z3, axis=1)

    one_scale(y0_ref, a0_ref, c0_ref, w0t_ref, mx0_ref, mn0_ref, st0_ref, _NS0)
    one_scale(y1_ref, a1_ref, c1_ref, w1t_ref, mx1_ref, mn1_ref, st1_ref, _NS1)


def _k5_body(mx0_ref, mn0_ref, mx1_ref, mn1_ref,
             a0_ref, c0_ref, a1_ref, c1_ref, o_ref):
    def one_scale(mx_ref, mn_ref, a_ref, c_ref, lo, hi):
        a = a_ref[...]
        pooled = jnp.where(a >= 0.0, mx_ref[...], mn_ref[...])
        o = jnp.maximum(pooled * a + c_ref[...], 0.0)
        o_ref[lo:hi, :] = jnp.transpose(o, (1, 0))

    one_scale(mx0_ref, mn0_ref, a0_ref, c0_ref, 0, 32)
    one_scale(mx1_ref, mn1_ref, a1_ref, c1_ref, 32, 96)


def _finalize(st, cnt, g, bta):
    mean = st[0] / cnt
    var = st[1] / cnt - mean * mean
    scale = g / jnp.sqrt(var + _EPS)
    return (scale[None, :], (bta - mean * scale)[None, :])


def kernel(points_xyz, features, indices,
           w0_0, g0_0, b0_0, w0_1, g0_1, b0_1,
           w1_0, g1_0, b1_0, w1_1, g1_1, b1_1):
    f32 = jnp.float32
    xtp = jnp.transpose(points_xyz, (0, 2, 1))[:, :, _PERM]

    grid_a = (_B, _M // _MBA)
    bd = np.zeros((_N, 64), np.float32)
    bd[_JJ, _JJ // 64] = 1.0
    nx, code, stops, tw0, tw1 = pl.pallas_call(
        _k1_body,
        grid=grid_a,
        in_specs=[
            pl.BlockSpec((None, _MBA, 1), lambda b, j: (b, j, 0)),
            pl.BlockSpec((None, _N, 3), lambda b, j: (b, 0, 0)),
            pl.BlockSpec((None, _CIN, _N), lambda b, j: (b, 0, 0)),
            pl.BlockSpec((None, 3, _N), lambda b, j: (b, 0, 0)),
            pl.BlockSpec((1, _N), lambda b, j: (0, 0)),
            pl.BlockSpec((_N, 64), lambda b, j: (0, 0)),
            pl.BlockSpec((19, 16), lambda b, j: (0, 0)),
            pl.BlockSpec((19, 32), lambda b, j: (0, 0)),
        ],
        out_specs=[
            pl.BlockSpec((None, _MBA, 3), lambda b, j: (b, j, 0)),
            pl.BlockSpec((None, _MBA, _N), lambda b, j: (b, j, 0)),
            pl.BlockSpec((None, _MBA, 1), lambda b, j: (b, j, 0)),
            pl.BlockSpec((None, _N, 16), lambda b, j: (b, 0, 0)),
            pl.BlockSpec((None, _N, 32), lambda b, j: (b, 0, 0)),
        ],
        out_shape=[
            jax.ShapeDtypeStruct((_B, _M, 3), f32),
            jax.ShapeDtypeStruct((_B, _M, _N), jnp.int8),
            jax.ShapeDtypeStruct((_B, _M, 1), jnp.int32),
            jax.ShapeDtypeStruct((_B, _N, 16), f32),
            jax.ShapeDtypeStruct((_B, _N, 32), f32),
        ],
    )(indices.reshape(_B, _M, 1), points_xyz, features, xtp,
      jnp.asarray(_PERM.astype(np.int32)[None, :]),
      jnp.asarray(bd), w0_0.T, w1_0.T)

    words = lax.bitcast_convert_type(
        code.reshape(_B * _M, _N // 4, 4), jnp.int32)

    mesh = plsc.VectorSubcoreMesh(core_axis_name="c", subcore_axis_name="s")
    y0r, y1r = pl.kernel(
        _sc_body,
        mesh=mesh,
        compiler_params=pltpu.CompilerParams(needs_layout_passes=False,
                                             use_tc_tiling_on_sc=False),
        out_type=[
            jax.ShapeDtypeStruct((_B * _M * _NS0, 16), f32),
            jax.ShapeDtypeStruct((_B * _M * _NS1, 32), f32),
        ],
        scratch_types=[
            pltpu.VMEM((_RB, _N // 4), jnp.int32),
            pltpu.VMEM((_RB,), jnp.int32),
            pltpu.VMEM((_RB * _NS0,), jnp.int32),
            pltpu.VMEM((_RB * _NS1,), jnp.int32),
            pltpu.VMEM((128, 16), f32),
            pltpu.VMEM((128, 32), f32),
            pltpu.SemaphoreType.DMA,
        ],
    )(words, stops.reshape(_NW, _RPW),
      tw0.reshape(_B * _N, 16), tw1.reshape(_B * _N, 32))

    y0r = y0r.reshape(_B, _M, _NS0, 16)
    y1r = y1r.reshape(_B, _M, _NS1, 32)

    grid_3 = (_B, _M // _MB3)
    y0, y1, st0, st1 = pl.pallas_call(
        _k3_body,
        grid=grid_3,
        in_specs=[
            pl.BlockSpec((None, _MB3, _NS0, 16), lambda b, j: (b, j, 0, 0)),
            pl.BlockSpec((None, _MB3, _NS1, 32), lambda b, j: (b, j, 0, 0)),
            pl.BlockSpec((None, _MB3, 3), lambda b, j: (b, j, 0)),
            pl.BlockSpec((19, 16), lambda b, j: (0, 0)),
            pl.BlockSpec((19, 32), lambda b, j: (0, 0)),
        ],
        out_specs=[
            pl.BlockSpec((None, _MB3, _NS0, 16), lambda b, j: (b, j, 0, 0)),
            pl.BlockSpec((None, _MB3, _NS1, 32), lambda b, j: (b, j, 0, 0)),
            pl.BlockSpec((2, 16), lambda b, j: (0, 0)),
            pl.BlockSpec((2, 32), lambda b, j: (0, 0)),
        ],
        out_shape=[
            jax.ShapeDtypeStruct((_B, _M, _NS0, 16), f32),
            jax.ShapeDtypeStruct((_B, _M, _NS1, 32), f32),
            jax.ShapeDtypeStruct((2, 16), f32),
            jax.ShapeDtypeStruct((2, 32), f32),
        ],
    )(y0r, y1r, nx, w0_0.T, w1_0.T)

    cnt0 = np.float32(_B * _M * _NS0)
    cnt1 = np.float32(_B * _M * _NS1)
    a0, c0 = _finalize(st0, cnt0, g0_0, b0_0)
    a1, c1 = _finalize(st1, cnt1, g1_0, b1_0)

    grid_b = (_B, _M // _MB2)
    mx0, mn0, mx1, mn1, st20, st21 = pl.pallas_call(
        _k4_body,
        grid=grid_b,
        in_specs=[
            pl.BlockSpec((None, _MB2, _NS0, 16), lambda b, j: (b, j, 0, 0)),
            pl.BlockSpec((None, _MB2, _NS1, 32), lambda b, j: (b, j, 0, 0)),
            pl.BlockSpec((1, 16), lambda b, j: (0, 0)),
            pl.BlockSpec((1, 16), lambda b, j: (0, 0)),
            pl.BlockSpec((1, 32), lambda b, j: (0, 0)),
            pl.BlockSpec((1, 32), lambda b, j: (0, 0)),
            pl.BlockSpec((16, 32), lambda b, j: (0, 0)),
            pl.BlockSpec((32, 64), lambda b, j: (0, 0)),
        ],
        out_specs=[
            pl.BlockSpec((None, _MB2, 32), lambda b, j: (b, j, 0)),
            pl.BlockSpec((None, _MB2, 32), lambda b, j: (b, j, 0)),
            pl.BlockSpec((None, _MB2, 64), lambda b, j: (b, j, 0)),
            pl.BlockSpec((None, _MB2, 64), lambda b, j: (b, j, 0)),
            pl.BlockSpec((2, 32), lambda b, j: (0, 0)),
            pl.BlockSpec((2, 64), lambda b, j: (0, 0)),
        ],
        out_shape=[
            jax.ShapeDtypeStruct((_B, _M, 32), f32),
            jax.ShapeDtypeStruct((_B, _M, 32), f32),
            jax.ShapeDtypeStruct((_B, _M, 64), f32),
            jax.ShapeDtypeStruct((_B, _M, 64), f32),
            jax.ShapeDtypeStruct((2, 32), f32),
            jax.ShapeDtypeStruct((2, 64), f32),
        ],
    )(y0, y1, a0, c0, a1, c1, w0_1.T, w1_1.T)

    a20, c20 = _finalize(st20, cnt0, g0_1, b0_1)
    a21, c21 = _finalize(st21, cnt1, g1_1, b1_1)

    new_features = pl.pallas_call(
        _k5_body,
        grid=(_B,),
        in_specs=[
            pl.BlockSpec((None, _M, 32), lambda b: (b, 0, 0)),
            pl.BlockSpec((None, _M, 32), lambda b: (b, 0, 0)),
            pl.BlockSpec((None, _M, 64), lambda b: (b, 0, 0)),
            pl.BlockSpec((None, _M, 64), lambda b: (b, 0, 0)),
            pl.BlockSpec((1, 32), lambda b: (0, 0)),
            pl.BlockSpec((1, 32), lambda b: (0, 0)),
            pl.BlockSpec((1, 64), lambda b: (0, 0)),
            pl.BlockSpec((1, 64), lambda b: (0, 0)),
        ],
        out_specs=pl.BlockSpec((None, 96, _M), lambda b: (b, 0, 0)),
        out_shape=jax.ShapeDtypeStruct((_B, 96, _M), f32),
    )(mx0, mn0, mx1, mn1, a20, c20, a21, c21)

    return (nx, new_features, indices)


# speedup vs baseline: 1.1587x; 1.0155x over previous
import functools

import jax
import jax.numpy as jnp
import numpy as np
from jax import lax
from jax.experimental import pallas as pl
from jax.experimental.pallas import tpu as pltpu
from jax.experimental.pallas import tpu_sc as plsc

_B, _N, _CIN = 8, 4096, 16
_M = 1024
_R0, _R1 = 0.2, 0.4
_NS0, _NS1 = 16, 32
_EPS = 1e-5
_MBA = 512
_MB3 = 128
_MB2 = 128

_NC, _NSC = 2, 16
_NW = _NC * _NSC
_RPW = (_B * _M) // _NW
_RB = 16
_NBATCH = _RPW // _RB
_WPR = _N // 4 // 16

_JJ = np.arange(_N)
_PERM = (_JJ // 64) * 64 + 16 * (_JJ % 4) + (_JJ % 64) // 4


def _cumsum_lanes(x):
    lane = jax.lax.broadcasted_iota(jnp.int32, x.shape, 1)
    s = 1
    while s < x.shape[1]:
        shifted = pltpu.roll(x, s, 1)
        x = x + jnp.where(lane >= s, shifted, 0)
        s *= 2
    return x


def _k1_body(idx_ref, xyz_ref, feat_ref, xtp_ref, perm_ref, bd_ref,
             w0t_ref, w1t_ref,
             nx_ref, code_ref, stop_ref, tw0_ref, tw1_ref):
    j = pl.program_id(1)
    xyz = xyz_ref[...]

    @pl.when(j == 0)
    def _():
        dn = (((0,), (0,)), ((), ()))
        tw0_ref[...] = (
            jnp.dot(xyz, w0t_ref[0:3, :], preferred_element_type=jnp.float32)
            + lax.dot_general(feat_ref[...], w0t_ref[3:19, :], dn,
                              preferred_element_type=jnp.float32))
        tw1_ref[...] = (
            jnp.dot(xyz, w1t_ref[0:3, :], preferred_element_type=jnp.float32)
            + lax.dot_general(feat_ref[...], w1t_ref[3:19, :], dn,
                              preferred_element_type=jnp.float32))

    oh = (idx_ref[...] == perm_ref[...]).astype(jnp.float32)
    q = jnp.concatenate(
        [jnp.sum(oh * xtp_ref[d:d + 1, :], axis=1, keepdims=True)
         for d in range(3)], axis=1)
    nx_ref[...] = q

    qn = jnp.sum(q * q, axis=1, keepdims=True)
    pn = jnp.sum(xtp_ref[...] * xtp_ref[...], axis=0, keepdims=True)
    qp = jnp.dot(q, xtp_ref[...], preferred_element_type=jnp.float32)
    d2 = qn + pn - 2.0 * qp
    valid0 = d2 < np.float32(_R0 * _R0)
    valid1 = d2 < np.float32(_R1 * _R1)
    code_ref[...] = jnp.where(valid0, 3, jnp.where(valid1, 1, 0)
                              ).astype(jnp.int8)

    pack = jnp.where(valid0, np.float32(65537.0),
                     jnp.where(valid1, np.float32(1.0), np.float32(0.0)))
    chk = jnp.dot(pack, bd_ref[...],
                  preferred_element_type=jnp.float32)
    cum = _cumsum_lanes(chk.astype(jnp.int32))
    c0ch = jax.lax.shift_right_logical(cum, 16)
    c1ch = jnp.bitwise_and(cum, 65535)
    short = jnp.logical_or(c0ch < _NS0, c1ch < _NS1)
    need = jnp.sum(jnp.where(short, 1, 0), axis=1, keepdims=True) + 1
    stop_ref[...] = jnp.minimum(need, _WPR)


def _sc_body(words, stops, tw0, tw1, y0, y1,
             wbuf, sbuf, ib0, ib1, rv0, rv1, sem):
    wid = lax.axis_index("s") * _NC + lax.axis_index("c")
    row0 = wid * _RPW
    lanes = lax.iota(jnp.int32, 16)

    def batch_body(bi, carry):
        rowbase = row0 + bi * _RB
        pltpu.sync_copy(words.at[pl.ds(rowbase, _RB), :], wbuf)
        pltpu.sync_copy(stops.at[wid, pl.ds(bi * _RB, _RB)], sbuf)

        def zero16(i, c):
            ib1[pl.ds(i * 16, 16)] = jnp.zeros((16,), jnp.int32)
            return c
        lax.fori_loop(0, _RB * _NS1 // 16, zero16, 0)

        def zero0(i, c):
            ib0[pl.ds(i * 16, 16)] = jnp.zeros((16,), jnp.int32)
            return c
        lax.fori_loop(0, _RB * _NS0 // 16, zero0, 0)

        def row_body(r, c):
            rowg = rowbase + r
            b = lax.shift_right_logical(rowg, 10)
            bn = b * _N

            sv = sbuf[...]
            nv = jnp.max(jnp.where(lanes == r, sv, 0))

            def step(v, st):
                c0, c1 = st
                w = wbuf[r, pl.ds(v * 16, 16)]
                for t in range(4):
                    fld = jnp.bitwise_and(
                        lax.shift_right_logical(w, 8 * t), 255)
                    idxv = 64 * v + 16 * t + lanes
                    m1 = fld >= 1
                    h1 = jnp.where(m1, 1, 0)
                    pos1 = plsc.cumsum(h1) - 1 + c1
                    plsc.store_scatter(
                        ib1, [r * _NS1 + pos1], idxv,
                        mask=jnp.logical_and(m1, pos1 < _NS1))
                    c1 = c1 + jnp.sum(h1)
                    m0 = fld >= 2
                    h0 = jnp.where(m0, 1, 0)
                    pos0 = plsc.cumsum(h0) - 1 + c0
                    plsc.store_scatter(
                        ib0, [r * _NS0 + pos0], idxv,
                        mask=jnp.logical_and(m0, pos0 < _NS0))
                    c0 = c0 + jnp.sum(h0)
                return (c0, c1)

            c0, c1 = lax.fori_loop(0, nv, step, (0, 0))

            cur0 = ib0[pl.ds(r * _NS0, 16)]
            f0 = jnp.max(jnp.where(lanes == 0, cur0, 0))
            ib0[pl.ds(r * _NS0, 16)] = jnp.where(lanes < c0, cur0, f0) + bn
            cur1a = ib1[pl.ds(r * _NS1, 16)]
            f1 = jnp.max(jnp.where(lanes == 0, cur1a, 0))
            ib1[pl.ds(r * _NS1, 16)] = (
                jnp.where(lanes < c1, cur1a, f1) + bn)
            cur1b = ib1[pl.ds(r * _NS1 + 16, 16)]
            ib1[pl.ds(r * _NS1 + 16, 16)] = (
                jnp.where(lanes + 16 < c1, cur1b, f1) + bn)
            return c

        lax.fori_loop(0, _RB, row_body, 0)

        for ch in range(_RB * _NS0 // 128):
            pltpu.async_copy(tw0.at[ib0.at[pl.ds(ch * 128, 128)]],
                             rv0, sem).wait()
            pltpu.sync_copy(
                rv0, y0.at[pl.ds(rowbase * _NS0 + ch * 128, 128), :])
        for ch in range(_RB * _NS1 // 128):
            pltpu.async_copy(tw1.at[ib1.at[pl.ds(ch * 128, 128)]],
                             rv1, sem).wait()
            pltpu.sync_copy(
                rv1, y1.at[pl.ds(rowbase * _NS1 + ch * 128, 128), :])
        return carry

    lax.fori_loop(0, _NBATCH, batch_body, 0)


def _k3_body(y0r_ref, y1r_ref, nx_ref, w0t_ref, w1t_ref,
             y0_ref, y1_ref, st0_ref, st1_ref):
    b = pl.program_id(0)
    j = pl.program_id(1)

    @pl.when(jnp.logical_and(b == 0, j == 0))
    def _():
        st0_ref[...] = jnp.zeros_like(st0_ref)
        st1_ref[...] = jnp.zeros_like(st1_ref)

    q = nx_ref[...]

    def one_scale(yr_ref, wt_ref, y_ref, st_ref):
        center = jnp.dot(q, wt_ref[0:3, :],
                         preferred_element_type=jnp.float32)
        y3 = yr_ref[...] - center[:, None, :]
        y_ref[...] = y3
        t = jnp.sum(y3, axis=1)
        st_ref[0:1, :] += jnp.sum(t, axis=0)[None, :]
        t2 = jnp.sum(y3 * y3, axis=1)
        st_ref[1:2, :] += jnp.sum(t2, axis=0)[None, :]

    one_scale(y0r_ref, w0t_ref, y0_ref, st0_ref)
    one_scale(y1r_ref, w1t_ref, y1_ref, st1_ref)


def _k4_body(y0_ref, y1_ref, a0_ref, c0_ref, a1_ref, c1_ref,
             w0t_ref, w1t_ref,
             mx0_ref, mn0_ref, mx1_ref, mn1_ref, st0_ref, st1_ref):
    b = pl.program_id(0)
    j = pl.program_id(1)

    @pl.when(jnp.logical_and(b == 0, j == 0))
    def _():
        st0_ref[...] = jnp.zeros_like(st0_ref)
        st1_ref[...] = jnp.zeros_like(st1_ref)

    def one_scale(y_ref, a_ref, c_ref, wt_ref, mx_ref, mn_ref, st_ref, ns):
        y = y_ref[...]
        h = jnp.maximum(y * a_ref[...][:, None, :] + c_ref[...][:, None, :],
                        0.0)
        c1 = h.shape[2]
        wt = wt_ref[...]
        z = jnp.dot(h.reshape(_MB2 * ns, c1), wt,
                    preferred_element_type=jnp.float32)
        st_ref[0:1, :] += jnp.sum(z, axis=0)[None, :]
        st_ref[1:2, :] += jnp.sum(z * z, axis=0)[None, :]
        z3 = z.reshape(_MB2, ns, wt.shape[1])
        mx_ref[...] = jnp.max(z3, axis=1)
        mn_ref[...] = jnp.min(z3, axis=1)

    one_scale(y0_ref, a0_ref, c0_ref, w0t_ref, mx0_ref, mn0_ref, st0_ref, _NS0)
    one_scale(y1_ref, a1_ref, c1_ref, w1t_ref, mx1_ref, mn1_ref, st1_ref, _NS1)


def _k5_body(mx0_ref, mn0_ref, mx1_ref, mn1_ref,
             a0_ref, c0_ref, a1_ref, c1_ref, o_ref):
    def one_scale(mx_ref, mn_ref, a_ref, c_ref, lo, hi):
        a = a_ref[...]
        pooled = jnp.where(a >= 0.0, mx_ref[...], mn_ref[...])
        o = jnp.maximum(pooled * a + c_ref[...], 0.0)
        o_ref[lo:hi, :] = jnp.transpose(o, (1, 0))

    one_scale(mx0_ref, mn0_ref, a0_ref, c0_ref, 0, 32)
    one_scale(mx1_ref, mn1_ref, a1_ref, c1_ref, 32, 96)


def _finalize(st, cnt, g, bta):
    mean = st[0] / cnt
    var = st[1] / cnt - mean * mean
    scale = g / jnp.sqrt(var + _EPS)
    return (scale[None, :], (bta - mean * scale)[None, :])


def kernel(points_xyz, features, indices,
           w0_0, g0_0, b0_0, w0_1, g0_1, b0_1,
           w1_0, g1_0, b1_0, w1_1, g1_1, b1_1):
    f32 = jnp.float32
    xtp = jnp.transpose(points_xyz, (0, 2, 1))[:, :, _PERM]

    grid_a = (_B, _M // _MBA)
    bd = np.zeros((_N, 64), np.float32)
    bd[_JJ, _JJ // 64] = 1.0
    nx, code, stops, tw0, tw1 = pl.pallas_call(
        _k1_body,
        grid=grid_a,
        in_specs=[
            pl.BlockSpec((None, _MBA, 1), lambda b, j: (b, j, 0)),
            pl.BlockSpec((None, _N, 3), lambda b, j: (b, 0, 0)),
            pl.BlockSpec((None, _CIN, _N), lambda b, j: (b, 0, 0)),
            pl.BlockSpec((None, 3, _N), lambda b, j: (b, 0, 0)),
            pl.BlockSpec((1, _N), lambda b, j: (0, 0)),
            pl.BlockSpec((_N, 64), lambda b, j: (0, 0)),
            pl.BlockSpec((19, 16), lambda b, j: (0, 0)),
            pl.BlockSpec((19, 32), lambda b, j: (0, 0)),
        ],
        out_specs=[
            pl.BlockSpec((None, _MBA, 3), lambda b, j: (b, j, 0)),
            pl.BlockSpec((None, _MBA, _N), lambda b, j: (b, j, 0)),
            pl.BlockSpec((None, _MBA, 1), lambda b, j: (b, j, 0)),
            pl.BlockSpec((None, _N, 16), lambda b, j: (b, 0, 0)),
            pl.BlockSpec((None, _N, 32), lambda b, j: (b, 0, 0)),
        ],
        out_shape=[
            jax.ShapeDtypeStruct((_B, _M, 3), f32),
            jax.ShapeDtypeStruct((_B, _M, _N), jnp.int8),
            jax.ShapeDtypeStruct((_B, _M, 1), jnp.int32),
            jax.ShapeDtypeStruct((_B, _N, 16), f32),
            jax.ShapeDtypeStruct((_B, _N, 32), f32),
        ],
    )(indices.reshape(_B, _M, 1), points_xyz, features, xtp,
      jnp.asarray(_PERM.astype(np.int32)[None, :]),
      jnp.asarray(bd), w0_0.T, w1_0.T)

    words = lax.bitcast_convert_type(
        code.reshape(_B * _M, _N // 4, 4), jnp.int32)

    mesh = plsc.VectorSubcoreMesh(core_axis_name="c", subcore_axis_name="s")
    y0r, y1r = pl.kernel(
        _sc_body,
        mesh=mesh,
        compiler_params=pltpu.CompilerParams(needs_layout_passes=False,
                                             use_tc_tiling_on_sc=False),
        out_type=[
            jax.ShapeDtypeStruct((_B * _M * _NS0, 16), f32),
            jax.ShapeDtypeStruct((_B * _M * _NS1, 32), f32),
        ],
        scratch_types=[
            pltpu.VMEM((_RB, _N // 4), jnp.int32),
            pltpu.VMEM((_RB,), jnp.int32),
            pltpu.VMEM((_RB * _NS0,), jnp.int32),
            pltpu.VMEM((_RB * _NS1,), jnp.int32),
            pltpu.VMEM((128, 16), f32),
            pltpu.VMEM((128, 32), f32),
            pltpu.SemaphoreType.DMA,
        ],
    )(words, stops.reshape(_NW, _RPW),
      tw0.reshape(_B * _N, 16), tw1.reshape(_B * _N, 32))

    y0r = y0r.reshape(_B, _M, _NS0, 16)
    y1r = y1r.reshape(_B, _M, _NS1, 32)

    grid_3 = (_B, _M // _MB3)
    y0, y1, st0, st1 = pl.pallas_call(
        _k3_body,
        grid=grid_3,
        in_specs=[
            pl.BlockSpec((None, _MB3, _NS0, 16), lambda b, j: (b, j, 0, 0)),
            pl.BlockSpec((None, _MB3, _NS1, 32), lambda b, j: (b, j, 0, 0)),
            pl.BlockSpec((None, _MB3, 3), lambda b, j: (b, j, 0)),
            pl.BlockSpec((19, 16), lambda b, j: (0, 0)),
            pl.BlockSpec((19, 32), lambda b, j: (0, 0)),
        ],
        out_specs=[
            pl.BlockSpec((None, _MB3, _NS0, 16), lambda b, j: (b, j, 0, 0)),
            pl.BlockSpec((None, _MB3, _NS1, 32), lambda b, j: (b, j, 0, 0)),
            pl.BlockSpec((2, 16), lambda b, j: (0, 0)),
            pl.BlockSpec((2, 32), lambda b, j: (0, 0)),
        ],
        out_shape=[
            jax.ShapeDtypeStruct((_B, _M, _NS0, 16), f32),
            jax.ShapeDtypeStruct((_B, _M, _NS1, 32), f32),
            jax.ShapeDtypeStruct((2, 16), f32),
            jax.ShapeDtypeStruct((2, 32), f32),
        ],
    )(y0r, y1r, nx, w0_0.T, w1_0.T)

    cnt0 = np.float32(_B * _M * _NS0)
    cnt1 = np.float32(_B * _M * _NS1)
    a0, c0 = _finalize(st0, cnt0, g0_0, b0_0)
    a1, c1 = _finalize(st1, cnt1, g1_0, b1_0)

    grid_b = (_B, _M // _MB2)
    mx0, mn0, mx1, mn1, st20, st21 = pl.pallas_call(
        _k4_body,
        grid=grid_b,
        in_specs=[
            pl.BlockSpec((None, _MB2, _NS0, 16), lambda b, j: (b, j, 0, 0)),
            pl.BlockSpec((None, _MB2, _NS1, 32), lambda b, j: (b, j, 0, 0)),
            pl.BlockSpec((1, 16), lambda b, j: (0, 0)),
            pl.BlockSpec((1, 16), lambda b, j: (0, 0)),
            pl.BlockSpec((1, 32), lambda b, j: (0, 0)),
            pl.BlockSpec((1, 32), lambda b, j: (0, 0)),
            pl.BlockSpec((16, 32), lambda b, j: (0, 0)),
            pl.BlockSpec((32, 64), lambda b, j: (0, 0)),
        ],
        out_specs=[
            pl.BlockSpec((None, _MB2, 32), lambda b, j: (b, j, 0)),
            pl.BlockSpec((None, _MB2, 32), lambda b, j: (b, j, 0)),
            pl.BlockSpec((None, _MB2, 64), lambda b, j: (b, j, 0)),
            pl.BlockSpec((None, _MB2, 64), lambda b, j: (b, j, 0)),
            pl.BlockSpec((2, 32), lambda b, j: (0, 0)),
            pl.BlockSpec((2, 64), lambda b, j: (0, 0)),
        ],
        out_shape=[
            jax.ShapeDtypeStruct((_B, _M, 32), f32),
            jax.ShapeDtypeStruct((_B, _M, 32), f32),
            jax.ShapeDtypeStruct((_B, _M, 64), f32),
            jax.ShapeDtypeStruct((_B, _M, 64), f32),
            jax.ShapeDtypeStruct((2, 32), f32),
            jax.ShapeDtypeStruct((2, 64), f32),
        ],
    )(y0, y1, a0, c0, a1, c1, w0_1.T, w1_1.T)

    a20, c20 = _finalize(st20, cnt0, g0_1, b0_1)
    a21, c21 = _finalize(st21, cnt1, g1_1, b1_1)

    new_features = pl.pallas_call(
        _k5_body,
        grid=(_B,),
        in_specs=[
            pl.BlockSpec((None, _M, 32), lambda b: (b, 0, 0)),
            pl.BlockSpec((None, _M, 32), lambda b: (b, 0, 0)),
            pl.BlockSpec((None, _M, 64), lambda b: (b, 0, 0)),
            pl.BlockSpec((None, _M, 64), lambda b: (b, 0, 0)),
            pl.BlockSpec((1, 32), lambda b: (0, 0)),
            pl.BlockSpec((1, 32), lambda b: (0, 0)),
            pl.BlockSpec((1, 64), lambda b: (0, 0)),
            pl.BlockSpec((1, 64), lambda b: (0, 0)),
        ],
        out_specs=pl.BlockSpec((None, 96, _M), lambda b: (b, 0, 0)),
        out_shape=jax.ShapeDtypeStruct((_B, 96, _M), f32),
    )(mx0, mn0, mx1, mn1, a20, c20, a21, c21)

    return (nx, new_features, indices)
